# baseline (speedup 1.0000x reference)
import functools

import jax
import jax.numpy as jnp
from jax import lax
from jax.experimental import pallas as pl
from jax.experimental.pallas import tpu as pltpu
from jax.experimental.pallas import tpu_sc as plsc

D = 128
L = 200
B = 1024
NC = 2
NS = 16
NW = NC * NS
ROWS_PER_W = B // NW
TOK_PER_W = ROWS_PER_W * L
CA = 128
CB = L - CA

_mesh = plsc.VectorSubcoreMesh(core_axis_name="c", subcore_axis_name="s")


@functools.partial(
    pl.kernel,
    mesh=_mesh,
    out_type=jax.ShapeDtypeStruct((B * L, D), jnp.float32),
    scratch_types=[
        pltpu.VMEM((TOK_PER_W,), jnp.int32),
        pltpu.VMEM((L, D), jnp.float32),
        pltpu.VMEM((L, D), jnp.float32),
        pltpu.VMEM((L, D), jnp.float32),
        pltpu.VMEM((L, D), jnp.float32),
        pltpu.SemaphoreType.DMA,
        pltpu.SemaphoreType.DMA,
        pltpu.SemaphoreType.DMA,
        pltpu.SemaphoreType.DMA,
        pltpu.SemaphoreType.DMA,
        pltpu.SemaphoreType.DMA,
        pltpu.SemaphoreType.DMA,
        pltpu.SemaphoreType.DMA,
        pltpu.SemaphoreType.DMA,
    ],
)
def _tok_pos_embed(x_hbm, tok_hbm, pos_hbm, out_hbm,
                   idx_all, pos_v, b0, b1, b2,
                   ga0, ga1, ga2, gb0, gb1, gb2, ss0, ss1, ss2):
    bufs = (b0, b1, b2)
    gasems = (ga0, ga1, ga2)
    gbsems = (gb0, gb1, gb2)
    ssems = (ss0, ss1, ss2)
    wid = lax.axis_index("s") * NC + lax.axis_index("c")
    tbase = wid * TOK_PER_W

    pltpu.sync_copy(x_hbm.at[pl.ds(tbase, TOK_PER_W)], idx_all)
    pltpu.sync_copy(pos_hbm, pos_v)

    def gstart(r, k):
        loc = r * L
        pltpu.async_copy(tok_hbm.at[idx_all.at[pl.ds(loc, CA)]],
                         bufs[k].at[pl.ds(0, CA)], gasems[k])
        pltpu.async_copy(tok_hbm.at[idx_all.at[pl.ds(loc + CA, CB)]],
                         bufs[k].at[pl.ds(CA, CB)], gbsems[k])

    def gwait_a(k):
        pltpu.make_async_copy(tok_hbm.at[idx_all.at[pl.ds(0, CA)]],
                              bufs[k].at[pl.ds(0, CA)], gasems[k]).wait()

    def gwait_b(k):
        pltpu.make_async_copy(tok_hbm.at[idx_all.at[pl.ds(0, CB)]],
                              bufs[k].at[pl.ds(CA, CB)], gbsems[k]).wait()

    def sstart(r, k):
        pltpu.async_copy(bufs[k], out_hbm.at[pl.ds(tbase + r * L, L)],
                         ssems[k])

    def swait(k):
        pltpu.make_async_copy(bufs[k], out_hbm.at[pl.ds(0, L)],
                              ssems[k]).wait()

    def add_half(k, start, rows):
        buf = bufs[k]

        def row_f(i, c):
            for dr in range(4):
                for j in range(D // 16):
                    plsc.addupdate(
                        buf.at[start + 4 * i + dr, pl.ds(j * 16, 16)],
                        pos_v[start + 4 * i + dr, pl.ds(j * 16, 16)])
            return c

        lax.fori_loop(0, rows // 4, row_f, 0)

    def visit(r, k, first=False, last_issue=False):
        gwait_a(k)
        add_half(k, 0, CA)
        if not first:
            swait((k + 2) % 3)
        if last_issue:
            @pl.when(r + 2 < ROWS_PER_W)
            def _():
                gstart(r + 2, (k + 2) % 3)
        else:
            gstart(r + 2, (k + 2) % 3)
        gwait_b(k)
        add_half(k, CA, CB)
        sstart(r, k)

    gstart(0, 0)
    gstart(1, 1)
    visit(0, 0, first=True)

    def body(t, carry):
        r = 3 * t + 1
        visit(r, 1)
        visit(r + 1, 2)
        visit(r + 2, 0, last_issue=True)
        return carry

    lax.fori_loop(0, (ROWS_PER_W - 2) // 3, body, 0)

    gwait_a(1)
    add_half(1, 0, CA)
    swait(0)
    gwait_b(1)
    add_half(1, CA, CB)
    sstart(ROWS_PER_W - 1, 1)
    swait(1)


def kernel(x, token_table, pos_table):
    x_flat = x.reshape(-1).astype(jnp.int32)
    out = _tok_pos_embed(x_flat, token_table, pos_table)
    return out.reshape(B, L, D)


# speedup vs baseline: 1.0051x; 1.0051x over previous
import functools

import jax
import jax.numpy as jnp
from jax import lax
from jax.experimental import pallas as pl
from jax.experimental.pallas import tpu as pltpu
from jax.experimental.pallas import tpu_sc as plsc

D = 128
L = 200
B = 1024
NC = 2
NS = 16
NW = NC * NS
ROWS_PER_W = B // NW
TOK_PER_W = ROWS_PER_W * L
CA = 128
CB = L - CA

_mesh = plsc.VectorSubcoreMesh(core_axis_name="c", subcore_axis_name="s")


@functools.partial(
    pl.kernel,
    mesh=_mesh,
    out_type=jax.ShapeDtypeStruct((B * L, D), jnp.float32),
    scratch_types=[
        pltpu.VMEM((TOK_PER_W,), jnp.int32),
        pltpu.VMEM((CA, D), jnp.float32),
        pltpu.VMEM((CB, D), jnp.float32),
        pltpu.VMEM((CA, D), jnp.float32),
        pltpu.VMEM((CB, D), jnp.float32),
        pltpu.VMEM((CA, D), jnp.float32),
        pltpu.VMEM((CB, D), jnp.float32),
        pltpu.VMEM((CA, D), jnp.float32),
        pltpu.VMEM((CB, D), jnp.float32),
        pltpu.SemaphoreType.DMA,
        pltpu.SemaphoreType.DMA,
        pltpu.SemaphoreType.DMA,
        pltpu.SemaphoreType.DMA,
        pltpu.SemaphoreType.DMA,
        pltpu.SemaphoreType.DMA,
        pltpu.SemaphoreType.DMA,
        pltpu.SemaphoreType.DMA,
        pltpu.SemaphoreType.DMA,
    ],
)
def _tok_pos_embed(x_hbm, tok_hbm, pos_hbm, out_hbm,
                   idx_all, pos_a, pos_b,
                   ba0, bb0, ba1, bb1, ba2, bb2,
                   ga0, ga1, ga2, gb0, gb1, gb2, ss0, ss1, ss2):
    bas = (ba0, ba1, ba2)
    bbs = (bb0, bb1, bb2)
    gasems = (ga0, ga1, ga2)
    gbsems = (gb0, gb1, gb2)
    ssems = (ss0, ss1, ss2)
    wid = lax.axis_index("s") * NC + lax.axis_index("c")
    tbase = wid * TOK_PER_W

    pltpu.sync_copy(x_hbm.at[pl.ds(tbase, TOK_PER_W)], idx_all)
    pltpu.sync_copy(pos_hbm.at[pl.ds(0, CA)], pos_a)
    pltpu.sync_copy(pos_hbm.at[pl.ds(CA, CB)], pos_b)

    def gstart(r, k):
        loc = r * L
        pltpu.async_copy(tok_hbm.at[idx_all.at[pl.ds(loc, CA)]],
                         bas[k], gasems[k])
        pltpu.async_copy(tok_hbm.at[idx_all.at[pl.ds(loc + CA, CB)]],
                         bbs[k], gbsems[k])

    def gwait_a(k):
        pltpu.make_async_copy(tok_hbm.at[idx_all.at[pl.ds(0, CA)]],
                              bas[k], gasems[k]).wait()

    def gwait_b(k):
        pltpu.make_async_copy(tok_hbm.at[idx_all.at[pl.ds(0, CB)]],
                              bbs[k], gbsems[k]).wait()

    def sstart_a(r, k):
        pltpu.async_copy(bas[k], out_hbm.at[pl.ds(tbase + r * L, CA)],
                         ssems[k])

    def sstart_b(r, k):
        pltpu.async_copy(bbs[k], out_hbm.at[pl.ds(tbase + r * L + CA, CB)],
                         ssems[k])

    def swait(k):
        pltpu.make_async_copy(bas[k], out_hbm.at[pl.ds(0, CA)],
                              ssems[k]).wait()
        pltpu.make_async_copy(bbs[k], out_hbm.at[pl.ds(0, CB)],
                              ssems[k]).wait()

    def add_half(buf, pos, rows):
        def row_f(i, c):
            for dr in range(4):
                for j in range(D // 16):
                    plsc.addupdate(buf.at[4 * i + dr, pl.ds(j * 16, 16)],
                                   pos[4 * i + dr, pl.ds(j * 16, 16)])
            return c

        lax.fori_loop(0, rows // 4, row_f, 0)

    def visit(r, k, first=False, last_issue=False):
        gwait_a(k)
        add_half(bas[k], pos_a, CA)
        sstart_a(r, k)
        if not first:
            swait((k + 2) % 3)
        if last_issue:
            @pl.when(r + 2 < ROWS_PER_W)
            def _():
                gstart(r + 2, (k + 2) % 3)
        else:
            gstart(r + 2, (k + 2) % 3)
        gwait_b(k)
        add_half(bbs[k], pos_b, CB)
        sstart_b(r, k)

    gstart(0, 0)
    gstart(1, 1)
    visit(0, 0, first=True)

    def body(t, carry):
        r = 3 * t + 1
        visit(r, 1)
        visit(r + 1, 2)
        visit(r + 2, 0, last_issue=True)
        return carry

    lax.fori_loop(0, (ROWS_PER_W - 2) // 3, body, 0)

    swait(0)
    gwait_a(1)
    add_half(bas[1], pos_a, CA)
    sstart_a(ROWS_PER_W - 1, 1)
    gwait_b(1)
    add_half(bbs[1], pos_b, CB)
    sstart_b(ROWS_PER_W - 1, 1)
    swait(1)


def kernel(x, token_table, pos_table):
    x_flat = x.reshape(-1).astype(jnp.int32)
    out = _tok_pos_embed(x_flat, token_table, pos_table)
    return out.reshape(B, L, D)
